# baseline (speedup 1.0000x reference)
import functools

import jax
import jax.numpy as jnp
from jax import lax
from jax.experimental import pallas as pl
from jax.experimental.pallas import tpu as pltpu
from jax.experimental.pallas import tpu_sc as plsc

_D = 64
_L = 20
_NC = 2
_NS = 16
_NW = _NC * _NS

_VOCAB = 1000000
_B, _H, _W = 1024, 5, 4
_N_CELLS = _B * _H * _W
_CPW = _N_CELLS // _NW
_C = 32
_NCH = _CPW // _C
_IPG = 128
_G = _C * _L // _IPG
_N_IDX = _N_CELLS * _L


def _sc_pool(idx_hbm, table_hbm, out_hbm, idx_v, rows_v, acc_v, sem):
    wid = lax.axis_index("s") * _NC + lax.axis_index("c")

    def chunk(ch, carry):
        cell_base = wid * _CPW + ch * _C
        idx_off = cell_base * _L
        pltpu.sync_copy(idx_hbm.at[pl.ds(idx_off, _C * _L)], idx_v)
        copies = []
        for j in range(_G):
            copies.append(
                pltpu.async_copy(
                    table_hbm.at[idx_v.at[pl.ds(j * _IPG, _IPG)]],
                    rows_v.at[pl.ds(j * _IPG, _IPG)],
                    sem,
                )
            )
        for cp in copies:
            cp.wait()

        def cell(c, carry2):
            r0 = c * _L
            for d in range(_D // 16):
                s = rows_v[r0, pl.ds(d * 16, 16)]
                for l in range(1, _L):
                    s = s + rows_v[r0 + l, pl.ds(d * 16, 16)]
                acc_v[c, pl.ds(d * 16, 16)] = s
            return carry2

        lax.fori_loop(0, _C, cell, 0, unroll=False)
        pltpu.sync_copy(acc_v, out_hbm.at[pl.ds(cell_base, _C)])
        return carry

    lax.fori_loop(0, _NCH, chunk, 0, unroll=False)


_sc_pool_call = functools.partial(
    pl.kernel,
    mesh=plsc.VectorSubcoreMesh(core_axis_name="c", subcore_axis_name="s"),
    out_type=jax.ShapeDtypeStruct((_N_CELLS, _D), jnp.float32),
    scratch_types=[
        pltpu.VMEM((_C * _L,), jnp.int32),
        pltpu.VMEM((_C * _L, 2 * _D), jnp.float32),
        pltpu.VMEM((_C, _D), jnp.float32),
        pltpu.SemaphoreType.DMA,
    ],
)(_sc_pool)


_SQRT1_2 = 0.7071067811865476


def _tc_head(sums_ref, wp_ref, bp_ref, o_ref):
    h = sums_ref[:] * (1.0 / _L)
    g = 0.5 * h * (1.0 + lax.erf(h * _SQRT1_2))
    w = wp_ref[0, :]
    o_ref[:] = jnp.sum(g * w[None, :], axis=1, keepdims=True) + bp_ref[0]


def kernel(x, table, Wp, bp):
    idx = (x.reshape(_N_IDX).astype(jnp.int32) >> 1)
    table_pairs = table.reshape(_VOCAB // 2, 2 * _D)
    sums = _sc_pool_call(idx, table_pairs)
    out = pl.pallas_call(
        _tc_head,
        grid=(_N_CELLS // 1024,),
        in_specs=[
            pl.BlockSpec((1024, _D), lambda i: (i, 0)),
            pl.BlockSpec((1, _D), lambda i: (0, 0)),
            pl.BlockSpec(memory_space=pltpu.SMEM),
        ],
        out_specs=pl.BlockSpec((1024, 1), lambda i: (i, 0)),
        out_shape=jax.ShapeDtypeStruct((_N_CELLS, 1), jnp.float32),
    )(sums, Wp, bp)
    return out.reshape(_B, _H, _W)


# speedup vs baseline: 1.0317x; 1.0317x over previous
import functools

import jax
import jax.numpy as jnp
from jax import lax
from jax.experimental import pallas as pl
from jax.experimental.pallas import tpu as pltpu
from jax.experimental.pallas import tpu_sc as plsc

_D = 64
_L = 20
_NC = 2
_NS = 16
_NW = _NC * _NS

_B, _H, _W = 1024, 5, 4
_HW = _H * _W
_NB = _B // 128
_N_CELLS = _B * _HW
_IPR = 128
_IDX_ROWS = _N_CELLS * _L // _IPR
_HWPW = _HW * _NB // _NW
_LG = 5
_NLG = _L // _LG


def _sc_pool(idx_hbm, table_hbm, out_hbm, idx_v, rows_v, acc_v, sem):
    wid = lax.axis_index("s") * _NC + lax.axis_index("c")
    tb = wid // 4
    wq = wid % 4

    def combo(i, carry):
        hw = wq * _HWPW + i
        h = hw // _W
        w = hw % _W

        def lgroup(lg, carry2):
            l0 = lg * _LG
            row0 = ((h * _L + l0) * _NB + tb) * _W + w
            for j in range(_LG):
                pltpu.sync_copy(idx_hbm.at[row0 + j * _NB * _W], idx_v.at[j])
            copies = []
            for j in range(_LG):
                copies.append(
                    pltpu.async_copy(
                        table_hbm.at[idx_v.at[j]], rows_v.at[j], sem
                    )
                )
            for cp in copies:
                cp.wait()

            def group(g, carry3):
                r0 = g * 8
                for t in range(8):
                    for d in range(_D // 16):
                        dd = pl.ds(d * 16, 16)
                        s = rows_v[0, r0 + t, dd]
                        for j in range(1, _LG):
                            s = s + rows_v[j, r0 + t, dd]
                        acc_v[r0 + t, dd] = jnp.where(
                            lg == 0, s, acc_v[r0 + t, dd] + s
                        )
                return carry3

            lax.fori_loop(0, _IPR // 8, group, 0, unroll=False)
            return carry2

        lax.fori_loop(0, _NLG, lgroup, 0, unroll=False)
        out_row = hw * _B + tb * _IPR
        pltpu.sync_copy(acc_v, out_hbm.at[pl.ds(out_row, _IPR)])
        return carry

    lax.fori_loop(0, _HWPW, combo, 0, unroll=False)


_sc_pool_call = functools.partial(
    pl.kernel,
    mesh=plsc.VectorSubcoreMesh(core_axis_name="c", subcore_axis_name="s"),
    out_type=jax.ShapeDtypeStruct((_N_CELLS, _D), jnp.float32),
    scratch_types=[
        pltpu.VMEM((_LG, _IPR), jnp.int32),
        pltpu.VMEM((_LG, _IPR, _D), jnp.float32),
        pltpu.VMEM((_IPR, _D), jnp.float32),
        pltpu.SemaphoreType.DMA,
    ],
    compiler_params=pltpu.CompilerParams(use_tc_tiling_on_sc=False),
)(_sc_pool)


_SQRT1_2 = 0.7071067811865476


def _tc_head(sums_ref, wp_ref, bp_ref, o_ref):
    h = sums_ref[:] * (1.0 / _L)
    g = 0.5 * h * (1.0 + lax.erf(h * _SQRT1_2))
    w = wp_ref[0, :]
    o_ref[:] = jnp.sum(g * w[None, :], axis=1, keepdims=True) + bp_ref[0]


def kernel(x, table, Wp, bp):
    xq = x.reshape(_NB, 128, _H, _W, _L).transpose(2, 4, 0, 3, 1)
    idx = lax.optimization_barrier(
        xq.reshape(_IDX_ROWS, _IPR).astype(jnp.int32)
    )
    sums = _sc_pool_call(idx, table)
    out = pl.pallas_call(
        _tc_head,
        grid=(_N_CELLS // 1024,),
        in_specs=[
            pl.BlockSpec((1024, _D), lambda i: (i, 0)),
            pl.BlockSpec((1, _D), lambda i: (0, 0)),
            pl.BlockSpec(memory_space=pltpu.SMEM),
        ],
        out_specs=pl.BlockSpec((1024, 1), lambda i: (i, 0)),
        out_shape=jax.ShapeDtypeStruct((_N_CELLS, 1), jnp.float32),
    )(sums, Wp, bp)
    return out.reshape(_H, _W, _B).transpose(2, 0, 1)


# speedup vs baseline: 1.0398x; 1.0078x over previous
import functools

import jax
import jax.numpy as jnp
from jax import lax
from jax.experimental import pallas as pl
from jax.experimental.pallas import tpu as pltpu
from jax.experimental.pallas import tpu_sc as plsc

_D = 64
_L = 20
_NC = 2
_NS = 16
_NW = _NC * _NS

_B, _H, _W = 1024, 5, 4
_N_CELLS = _B * _H * _W
_IPR = 128
_IDX_ROWS = _N_CELLS * _L // _IPR
_RPC = 5
_CELLS_PC = _RPC * _IPR // _L
_RPW = _IDX_ROWS // _NW
_NCH = _RPW // _RPC


def _sc_pool(idx_hbm, table_hbm, out_hbm, idx_v, rows_v, out_v, sem):
    wid = lax.axis_index("s") * _NC + lax.axis_index("c")
    row_base = wid * _RPW
    cell_base = wid * (_RPW * _IPR // _L)

    def chunk(k, carry):
        r0 = row_base + k * _RPC
        pltpu.sync_copy(idx_hbm.at[pl.ds(r0, _RPC)], idx_v)
        copies = []
        for j in range(_RPC):
            copies.append(
                pltpu.async_copy(
                    table_hbm.at[idx_v.at[j]],
                    rows_v.at[pl.ds(j * _IPR, _IPR)],
                    sem,
                )
            )
        for cp in copies:
            cp.wait()

        def cell(c, carry2):
            t0 = c * _L
            for d in range(_D // 16):
                dd = pl.ds(d * 16, 16)
                s = rows_v[t0, dd]
                for t in range(1, _L):
                    s = s + rows_v[t0 + t, dd]
                out_v[c, dd] = s
            return carry2

        lax.fori_loop(0, _CELLS_PC, cell, 0, unroll=False)
        pltpu.sync_copy(
            out_v, out_hbm.at[pl.ds(cell_base + k * _CELLS_PC, _CELLS_PC)]
        )
        return carry

    lax.fori_loop(0, _NCH, chunk, 0, unroll=False)


_sc_pool_call = functools.partial(
    pl.kernel,
    mesh=plsc.VectorSubcoreMesh(core_axis_name="c", subcore_axis_name="s"),
    out_type=jax.ShapeDtypeStruct((_N_CELLS, _D), jnp.float32),
    scratch_types=[
        pltpu.VMEM((_RPC, _IPR), jnp.int32),
        pltpu.VMEM((_RPC * _IPR, _D), jnp.float32),
        pltpu.VMEM((_CELLS_PC, _D), jnp.float32),
        pltpu.SemaphoreType.DMA,
    ],
    compiler_params=pltpu.CompilerParams(use_tc_tiling_on_sc=False),
)(_sc_pool)


_SQRT1_2 = 0.7071067811865476


def _tc_head(sums_ref, wp_ref, bp_ref, o_ref):
    h = sums_ref[:] * (1.0 / _L)
    g = 0.5 * h * (1.0 + lax.erf(h * _SQRT1_2))
    w = wp_ref[0, :]
    o_ref[:] = jnp.sum(g * w[None, :], axis=1, keepdims=True) + bp_ref[0]


def kernel(x, table, Wp, bp):
    idx = x.reshape(_IDX_ROWS, _IPR).astype(jnp.int32)
    sums = _sc_pool_call(idx, table)
    out = pl.pallas_call(
        _tc_head,
        grid=(_N_CELLS // 1024,),
        in_specs=[
            pl.BlockSpec((1024, _D), lambda i: (i, 0)),
            pl.BlockSpec((1, _D), lambda i: (0, 0)),
            pl.BlockSpec(memory_space=pltpu.SMEM),
        ],
        out_specs=pl.BlockSpec((1024, 1), lambda i: (i, 0)),
        out_shape=jax.ShapeDtypeStruct((_N_CELLS, 1), jnp.float32),
    )(sums, Wp, bp)
    return out.reshape(_B, _H, _W)


# speedup vs baseline: 1.7202x; 1.6544x over previous
import functools

import jax
import jax.numpy as jnp
from jax import lax
from jax.experimental import pallas as pl
from jax.experimental.pallas import tpu as pltpu
from jax.experimental.pallas import tpu_sc as plsc

_V = 1000000
_D = 64
_L = 20
_NC = 2
_NS = 16
_NW = _NC * _NS

_B, _H, _W = 1024, 5, 4
_N_CELLS = _B * _H * _W
_IPR = 128
_IDX_ROWS = _N_CELLS * _L // _IPR
_RPC = 5
_CELLS_PC = _RPC * _IPR // _L
_RPW = _IDX_ROWS // _NW
_NCH = _RPW // _RPC

_TCOLS = 6400


def _tc_repack(tt_ref, o_ref, at_ref):
    at_ref[:] = tt_ref[:].T
    o_ref[:] = jnp.concatenate([at_ref[0::2, :], at_ref[1::2, :]], axis=1)


def _sc_pool(idx_hbm, table_hbm, out_hbm, idx_v, rows_v, out_v, sem):
    wid = lax.axis_index("s") * _NC + lax.axis_index("c")
    row_base = wid * _RPW
    cell_base = wid * (_RPW * _IPR // _L)

    def chunk(k, carry):
        r0 = row_base + k * _RPC
        pltpu.sync_copy(idx_hbm.at[pl.ds(r0, _RPC)], idx_v)
        copies = []
        for j in range(_RPC):
            copies.append(
                pltpu.async_copy(
                    table_hbm.at[idx_v.at[j]],
                    rows_v.at[pl.ds(j * _IPR, _IPR)],
                    sem,
                )
            )
        for cp in copies:
            cp.wait()

        def cell(c, carry2):
            t0 = c * _L
            for d in range(_D // 16):
                dd = pl.ds(d * 16, 16)
                s = rows_v[t0, dd]
                for t in range(1, _L):
                    s = s + rows_v[t0 + t, dd]
                out_v[c, dd] = s
            return carry2

        lax.fori_loop(0, _CELLS_PC, cell, 0, unroll=False)
        pltpu.sync_copy(
            out_v, out_hbm.at[pl.ds(cell_base + k * _CELLS_PC, _CELLS_PC)]
        )
        return carry

    lax.fori_loop(0, _NCH, chunk, 0, unroll=False)


_sc_pool_call = functools.partial(
    pl.kernel,
    mesh=plsc.VectorSubcoreMesh(core_axis_name="c", subcore_axis_name="s"),
    out_type=jax.ShapeDtypeStruct((_N_CELLS, _D), jnp.float32),
    scratch_types=[
        pltpu.VMEM((_RPC, _IPR), jnp.int32),
        pltpu.VMEM((_RPC * _IPR, _D), jnp.float32),
        pltpu.VMEM((_CELLS_PC, _D), jnp.float32),
        pltpu.SemaphoreType.DMA,
    ],
    compiler_params=pltpu.CompilerParams(use_tc_tiling_on_sc=False),
)(_sc_pool)


_SQRT1_2 = 0.7071067811865476


def _tc_head(sums_ref, wp_ref, bp_ref, o_ref):
    h = sums_ref[:] * (1.0 / _L)
    g = 0.5 * h * (1.0 + lax.erf(h * _SQRT1_2))
    w = wp_ref[0, :]
    o_ref[:] = jnp.sum(g * w[None, :], axis=1, keepdims=True) + bp_ref[0]


def kernel(x, table, Wp, bp):
    packed = pl.pallas_call(
        _tc_repack,
        grid=(pl.cdiv(_V, _TCOLS),),
        in_specs=[pl.BlockSpec((_D, _TCOLS), lambda i: (0, i))],
        out_specs=pl.BlockSpec((_TCOLS // 2, 2 * _D), lambda i: (i, 0)),
        out_shape=jax.ShapeDtypeStruct((_V // 2, 2 * _D), jnp.float32),
        scratch_shapes=[pltpu.VMEM((_TCOLS, _D), jnp.float32)],
    )(table.T)
    table_lin = packed.reshape(_V, _D)
    idx = x.reshape(_IDX_ROWS, _IPR).astype(jnp.int32)
    sums = _sc_pool_call(idx, table_lin)
    out = pl.pallas_call(
        _tc_head,
        grid=(_N_CELLS // 1024,),
        in_specs=[
            pl.BlockSpec((1024, _D), lambda i: (i, 0)),
            pl.BlockSpec((1, _D), lambda i: (0, 0)),
            pl.BlockSpec(memory_space=pltpu.SMEM),
        ],
        out_specs=pl.BlockSpec((1024, 1), lambda i: (i, 0)),
        out_shape=jax.ShapeDtypeStruct((_N_CELLS, 1), jnp.float32),
    )(sums, Wp, bp)
    return out.reshape(_B, _H, _W)
